# Initial kernel scaffold; baseline (speedup 1.0000x reference)
import jax
import jax.numpy as jnp
from jax.experimental import pallas as pl


def kernel(x, edge_index, batch, W0, b0, W1, b1, W2, b2, Wout, bout):
    raise NotImplementedError("write your pallas kernel here")


# speedup vs baseline: 24.6357x; 24.6357x over previous
import functools

import jax
import jax.numpy as jnp
from jax import lax
from jax.experimental import pallas as pl
from jax.experimental.pallas import tpu as pltpu
from jax.experimental.pallas import tpu_sc as plsc

NC = 2
NS = 16
NW = NC * NS
CHUNK = 125
IB = 16
NPAD = 10112
STRIPE = NPAD // NS
ZROWS = 8


def _sc_mesh():
    return plsc.VectorSubcoreMesh(core_axis_name="c", subcore_axis_name="s",
                                  num_cores=NC, num_subcores=NS)


def _make_deg_kernel(nchunk):
    @functools.partial(
        pl.kernel,
        out_type=jax.ShapeDtypeStruct((NC, NPAD, 16), jnp.float32),
        mesh=_sc_mesh(),
        scratch_types=[
            pltpu.VMEM((nchunk, CHUNK), jnp.int32),
            pltpu.VMEM((ZROWS, 16), jnp.float32),
            pltpu.VMEM((CHUNK, 16), jnp.float32),
            pltpu.VMEM_SHARED((NPAD, 16), jnp.float32),
        ],
    )
    def deg_k(dst_hbm, out_hbm, dst_v, zbuf, ones, acc):
        cid = lax.axis_index("c")
        sid = lax.axis_index("s")
        wid = sid * NC + cid

        def zrow(r, _):
            zbuf[r, :] = jnp.zeros((16,), jnp.float32)
            return 0
        lax.fori_loop(0, ZROWS, zrow, 0)

        def zcopy(k, _):
            pltpu.sync_copy(zbuf, acc.at[pl.ds(sid * STRIPE + k * ZROWS, ZROWS)])
            return 0
        lax.fori_loop(0, STRIPE // ZROWS, zcopy, 0)

        def orow(r, _):
            ones[r, :] = jnp.ones((16,), jnp.float32)
            return 0
        lax.fori_loop(0, CHUNK, orow, 0)

        pltpu.sync_copy(dst_hbm.at[wid], dst_v)
        plsc.subcore_barrier()

        def body(j, _):
            pltpu.sync_copy(ones, acc.at[dst_v.at[j]], add=True)
            return 0
        lax.fori_loop(0, nchunk, body, 0)

        plsc.subcore_barrier()
        pltpu.sync_copy(acc.at[pl.ds(sid * STRIPE, STRIPE)],
                        out_hbm.at[cid, pl.ds(sid * STRIPE, STRIPE)])

    return deg_k


def _make_scatter_kernel(d, nchunk):
    ngroup = nchunk // IB

    @functools.partial(
        pl.kernel,
        out_type=jax.ShapeDtypeStruct((NC, NPAD, d), jnp.float32),
        mesh=_sc_mesh(),
        scratch_types=[
            pltpu.VMEM((IB, CHUNK), jnp.int32),
            pltpu.VMEM((IB, CHUNK), jnp.int32),
            pltpu.VMEM((ZROWS, d), jnp.float32),
            pltpu.VMEM((CHUNK, d), jnp.float32),
            pltpu.VMEM((CHUNK, d), jnp.float32),
            pltpu.VMEM_SHARED((NPAD, d), jnp.float32),
            pltpu.SemaphoreType.DMA,
            pltpu.SemaphoreType.DMA,
        ],
    )
    def scatter_k(src_hbm, dst_hbm, g_hbm, out_hbm,
                  src_v, dst_v, zbuf, bufa, bufb, acc, sema, semb):
        cid = lax.axis_index("c")
        sid = lax.axis_index("s")
        wid = sid * NC + cid

        def zrow(r, _):
            for c in range(d // 16):
                zbuf[r, pl.ds(c * 16, 16)] = jnp.zeros((16,), jnp.float32)
            return 0
        lax.fori_loop(0, ZROWS, zrow, 0)

        def zcopy(k, _):
            pltpu.sync_copy(zbuf, acc.at[pl.ds(sid * STRIPE + k * ZROWS, ZROWS)])
            return 0
        lax.fori_loop(0, STRIPE // ZROWS, zcopy, 0)
        plsc.subcore_barrier()

        def group(gi, _):
            pltpu.sync_copy(src_hbm.at[wid, pl.ds(gi * IB, IB)], src_v)
            pltpu.sync_copy(dst_hbm.at[wid, pl.ds(gi * IB, IB)], dst_v)
            pltpu.async_copy(g_hbm.at[src_v.at[0]], bufa, sema)

            def body(jj, _):
                j = jj * 2
                pltpu.async_copy(g_hbm.at[src_v.at[j + 1]], bufb, semb)
                pltpu.make_async_copy(g_hbm.at[src_v.at[j]], bufa, sema).wait()
                pltpu.sync_copy(bufa, acc.at[dst_v.at[j]], add=True)

                @pl.when(jj + 1 < IB // 2)
                def _():
                    pltpu.async_copy(g_hbm.at[src_v.at[j + 2]], bufa, sema)

                pltpu.make_async_copy(g_hbm.at[src_v.at[j + 1]], bufb,
                                      semb).wait()
                pltpu.sync_copy(bufb, acc.at[dst_v.at[j + 1]], add=True)
                return 0
            lax.fori_loop(0, IB // 2, body, 0)
            return 0
        lax.fori_loop(0, ngroup, group, 0)

        plsc.subcore_barrier()
        pltpu.sync_copy(acc.at[pl.ds(sid * STRIPE, STRIPE)],
                        out_hbm.at[cid, pl.ds(sid * STRIPE, STRIPE)])

    return scatter_k


_BLK = 1000


def _tc_in_body(x, dinv, w, o):
    o[...] = dinv[...] * jnp.dot(x[...], w[...],
                                 preferred_element_type=jnp.float32)


def _tc_mid_body(p0, p1, g, dinv, b, w, o):
    h = dinv[...] * (p0[...] + p1[...] + g[...]) + b[...]
    h = jnp.maximum(h, 0.0)
    o[...] = dinv[...] * jnp.dot(h, w[...], preferred_element_type=jnp.float32)


def _tc_out_body(p0, p1, g, dinv, b, w, bout, o):
    h = dinv[...] * (p0[...] + p1[...] + g[...]) + b[...]
    h = jnp.maximum(h, 0.0)
    logits = jnp.dot(h, w[...], preferred_element_type=jnp.float32) + bout[...]
    m = jnp.max(logits, axis=1, keepdims=True)
    e = jnp.exp(logits - m)
    o[...] = e / jnp.sum(e, axis=1, keepdims=True)


def _row_spec(d):
    return pl.BlockSpec((_BLK, d), lambda i: (i, 0))


def _full_spec(r, c):
    return pl.BlockSpec((r, c), lambda i: (0, 0))


def kernel(x, edge_index, batch, W0, b0, W1, b1, W2, b2, Wout, bout):
    n, d = x.shape
    h = W0.shape[1]
    e = edge_index.shape[1]
    nchunk = e // NW // CHUNK

    src3 = edge_index[0].reshape(NW, nchunk, CHUNK)
    dst3 = edge_index[1].reshape(NW, nchunk, CHUNK)

    deg_k = _make_deg_kernel(nchunk)
    scat_k = _make_scatter_kernel(h, nchunk)

    degp = deg_k(dst3)
    deg = degp[0, :n, 0] + degp[1, :n, 0] + 1.0
    dinv = lax.rsqrt(deg).reshape(n, 1)

    t_in = pl.pallas_call(
        _tc_in_body,
        grid=(n // _BLK,),
        in_specs=[_row_spec(d), _row_spec(1), _full_spec(d, h)],
        out_specs=_row_spec(h),
        out_shape=jax.ShapeDtypeStruct((n, h), jnp.float32),
    )

    t_mid = pl.pallas_call(
        _tc_mid_body,
        grid=(n // _BLK,),
        in_specs=[_row_spec(h), _row_spec(h), _row_spec(h), _row_spec(1),
                  _full_spec(1, h), _full_spec(h, h)],
        out_specs=_row_spec(h),
        out_shape=jax.ShapeDtypeStruct((n, h), jnp.float32),
    )

    t_out = pl.pallas_call(
        _tc_out_body,
        grid=(n // _BLK,),
        in_specs=[_row_spec(h), _row_spec(h), _row_spec(h), _row_spec(1),
                  _full_spec(1, h), _full_spec(h, d), _full_spec(1, d)],
        out_specs=_row_spec(d),
        out_shape=jax.ShapeDtypeStruct((n, d), jnp.float32),
    )

    g = t_in(x, dinv, W0)
    p = scat_k(src3, dst3, g)
    g = t_mid(p[0, :n], p[1, :n], g, dinv, b0.reshape(1, h), W1)
    p = scat_k(src3, dst3, g)
    g = t_mid(p[0, :n], p[1, :n], g, dinv, b1.reshape(1, h), W2)
    p = scat_k(src3, dst3, g)
    return t_out(p[0, :n], p[1, :n], g, dinv, b2.reshape(1, h), Wout,
                 bout.reshape(1, d))
